# baseline (device time: 10048 ns/iter reference)
import jax
import jax.numpy as jnp
from jax import lax
from jax.experimental import pallas as pl
from jax.experimental.pallas import tpu as pltpu

N_CHUNKS = 8


def kernel(x):
    m, n = x.shape
    chunk_m = m // N_CHUNKS

    def chunk_copy(x_hbm, buf, copy_sems, i):
        return pltpu.make_async_copy(
            x_hbm.at[pl.ds(i * chunk_m, chunk_m), :],
            buf.at[i % 2],
            copy_sems.at[i % 2],
        )

    def body(x_hbm, out_hbm, buf, acc, recv_buf, out_vmem, copy_sems, send_sem,
             recv_sem, out_copy_sem):
        my_x = lax.axis_index("x")
        my_y = lax.axis_index("y")
        nbr = (1 - my_x, my_y)

        chunk_copy(x_hbm, buf, copy_sems, 0).start()

        barrier_sem = pltpu.get_barrier_semaphore()
        pl.semaphore_signal(
            barrier_sem, inc=1, device_id=nbr,
            device_id_type=pl.DeviceIdType.MESH,
        )

        for i in range(N_CHUNKS):
            if i + 1 < N_CHUNKS:
                chunk_copy(x_hbm, buf, copy_sems, i + 1).start()
            chunk_copy(x_hbm, buf, copy_sems, i).wait()
            chunk_max = jnp.max(buf[i % 2], axis=0, keepdims=True)
            if i == 0:
                acc[:, :] = chunk_max
            else:
                acc[:, :] = jnp.maximum(acc[:, :], chunk_max)

        pl.semaphore_wait(barrier_sem, 1)

        rdma = pltpu.make_async_remote_copy(
            src_ref=acc,
            dst_ref=recv_buf,
            send_sem=send_sem,
            recv_sem=recv_sem,
            device_id=nbr,
            device_id_type=pl.DeviceIdType.MESH,
        )
        rdma.start()
        rdma.wait()

        out_vmem[:, :] = jnp.maximum(acc[:, :], recv_buf[:, :])
        out_cp = pltpu.make_async_copy(out_vmem, out_hbm, out_copy_sem)
        out_cp.start()
        out_cp.wait()

    return pl.pallas_call(
        body,
        out_shape=jax.ShapeDtypeStruct((1, n), x.dtype),
        in_specs=[pl.BlockSpec(memory_space=pltpu.MemorySpace.HBM)],
        out_specs=pl.BlockSpec(memory_space=pltpu.MemorySpace.HBM),
        scratch_shapes=[
            pltpu.VMEM((2, m // N_CHUNKS, n), x.dtype),
            pltpu.VMEM((1, n), x.dtype),
            pltpu.VMEM((1, n), x.dtype),
            pltpu.VMEM((1, n), x.dtype),
            pltpu.SemaphoreType.DMA((2,)),
            pltpu.SemaphoreType.DMA,
            pltpu.SemaphoreType.DMA,
            pltpu.SemaphoreType.DMA,
        ],
        compiler_params=pltpu.CompilerParams(collective_id=0),
    )(pltpu.with_memory_space_constraint(x, pltpu.MemorySpace.HBM))


# device time: 7844 ns/iter; 1.2810x vs baseline; 1.2810x over previous
import jax
import jax.numpy as jnp
from jax import lax
from jax.experimental import pallas as pl
from jax.experimental.pallas import tpu as pltpu

N_CHUNKS = 8


def kernel(x):
    m, n = x.shape
    chunk_m = m // N_CHUNKS

    def chunk_copy(x_hbm, buf, copy_sems, i):
        return pltpu.make_async_copy(
            x_hbm.at[pl.ds(i * chunk_m, chunk_m), :],
            buf.at[i],
            copy_sems.at[i],
        )

    def body(x_hbm, out_hbm, buf, acc, recv_buf, out_vmem, copy_sems, send_sem,
             recv_sem, out_copy_sem):
        my_x = lax.axis_index("x")
        my_y = lax.axis_index("y")
        nbr = (1 - my_x, my_y)

        for i in range(N_CHUNKS):
            chunk_copy(x_hbm, buf, copy_sems, i).start()

        barrier_sem = pltpu.get_barrier_semaphore()
        pl.semaphore_signal(
            barrier_sem, inc=1, device_id=nbr,
            device_id_type=pl.DeviceIdType.MESH,
        )

        acc_val = None
        for i in range(N_CHUNKS):
            chunk_copy(x_hbm, buf, copy_sems, i).wait()
            chunk_max = jnp.max(buf[i], axis=0, keepdims=True)
            acc_val = chunk_max if acc_val is None else jnp.maximum(acc_val, chunk_max)
        acc[:, :] = acc_val

        pl.semaphore_wait(barrier_sem, 1)

        rdma = pltpu.make_async_remote_copy(
            src_ref=acc,
            dst_ref=recv_buf,
            send_sem=send_sem,
            recv_sem=recv_sem,
            device_id=nbr,
            device_id_type=pl.DeviceIdType.MESH,
        )
        rdma.start()
        rdma.wait()

        out_vmem[:, :] = jnp.maximum(acc[:, :], recv_buf[:, :])
        out_cp = pltpu.make_async_copy(out_vmem, out_hbm, out_copy_sem)
        out_cp.start()
        out_cp.wait()

    return pl.pallas_call(
        body,
        out_shape=jax.ShapeDtypeStruct((1, n), x.dtype),
        in_specs=[pl.BlockSpec(memory_space=pltpu.MemorySpace.HBM)],
        out_specs=pl.BlockSpec(memory_space=pltpu.MemorySpace.HBM),
        scratch_shapes=[
            pltpu.VMEM((N_CHUNKS, m // N_CHUNKS, n), x.dtype),
            pltpu.VMEM((1, n), x.dtype),
            pltpu.VMEM((1, n), x.dtype),
            pltpu.VMEM((1, n), x.dtype),
            pltpu.SemaphoreType.DMA((N_CHUNKS,)),
            pltpu.SemaphoreType.DMA,
            pltpu.SemaphoreType.DMA,
            pltpu.SemaphoreType.DMA,
        ],
        compiler_params=pltpu.CompilerParams(collective_id=0),
    )(pltpu.with_memory_space_constraint(x, pltpu.MemorySpace.HBM))


# device time: 7835 ns/iter; 1.2825x vs baseline; 1.0011x over previous
import jax
import jax.numpy as jnp
from jax import lax
from jax.experimental import pallas as pl
from jax.experimental.pallas import tpu as pltpu

N_CHUNKS = 4


def kernel(x):
    m, n = x.shape
    chunk_m = m // N_CHUNKS

    def chunk_copy(x_hbm, buf, copy_sems, i):
        return pltpu.make_async_copy(
            x_hbm.at[pl.ds(i * chunk_m, chunk_m), :],
            buf.at[i],
            copy_sems.at[i],
        )

    def body(x_hbm, out_hbm, buf, acc, recv_buf, out_vmem, copy_sems, send_sem,
             recv_sem, out_copy_sem):
        my_x = lax.axis_index("x")
        my_y = lax.axis_index("y")
        nbr = (1 - my_x, my_y)

        for i in range(N_CHUNKS):
            chunk_copy(x_hbm, buf, copy_sems, i).start()

        barrier_sem = pltpu.get_barrier_semaphore()
        pl.semaphore_signal(
            barrier_sem, inc=1, device_id=nbr,
            device_id_type=pl.DeviceIdType.MESH,
        )

        acc_val = None
        for i in range(N_CHUNKS):
            chunk_copy(x_hbm, buf, copy_sems, i).wait()
            chunk_max = jnp.max(buf[i], axis=0, keepdims=True)
            acc_val = chunk_max if acc_val is None else jnp.maximum(acc_val, chunk_max)
        acc[:, :] = acc_val

        pl.semaphore_wait(barrier_sem, 1)

        rdma = pltpu.make_async_remote_copy(
            src_ref=acc,
            dst_ref=recv_buf,
            send_sem=send_sem,
            recv_sem=recv_sem,
            device_id=nbr,
            device_id_type=pl.DeviceIdType.MESH,
        )
        rdma.start()
        rdma.wait()

        out_vmem[:, :] = jnp.maximum(acc[:, :], recv_buf[:, :])
        out_cp = pltpu.make_async_copy(out_vmem, out_hbm, out_copy_sem)
        out_cp.start()
        out_cp.wait()

    return pl.pallas_call(
        body,
        out_shape=jax.ShapeDtypeStruct((1, n), x.dtype),
        in_specs=[pl.BlockSpec(memory_space=pltpu.MemorySpace.HBM)],
        out_specs=pl.BlockSpec(memory_space=pltpu.MemorySpace.HBM),
        scratch_shapes=[
            pltpu.VMEM((N_CHUNKS, m // N_CHUNKS, n), x.dtype),
            pltpu.VMEM((1, n), x.dtype),
            pltpu.VMEM((1, n), x.dtype),
            pltpu.VMEM((1, n), x.dtype),
            pltpu.SemaphoreType.DMA((N_CHUNKS,)),
            pltpu.SemaphoreType.DMA,
            pltpu.SemaphoreType.DMA,
            pltpu.SemaphoreType.DMA,
        ],
        compiler_params=pltpu.CompilerParams(collective_id=0),
    )(pltpu.with_memory_space_constraint(x, pltpu.MemorySpace.HBM))


# device time: 7821 ns/iter; 1.2847x vs baseline; 1.0018x over previous
import jax
import jax.numpy as jnp
from jax import lax
from jax.experimental import pallas as pl
from jax.experimental.pallas import tpu as pltpu

N_CHUNKS = 4


def kernel(x):
    m, n = x.shape
    chunk_m = m // N_CHUNKS

    def chunk_copy(x_hbm, buf, copy_sems, i):
        return pltpu.make_async_copy(
            x_hbm.at[pl.ds(i * chunk_m, chunk_m), :],
            buf.at[i],
            copy_sems.at[i],
        )

    def body(x_hbm, out_hbm, buf, acc, recv_buf, out_vmem, copy_sems, send_sem,
             recv_sem, out_copy_sem):
        my_x = lax.axis_index("x")
        my_y = lax.axis_index("y")
        nbr = (1 - my_x, my_y)

        for i in range(N_CHUNKS):
            chunk_copy(x_hbm, buf, copy_sems, i).start()

        barrier_sem = pltpu.get_barrier_semaphore()
        pl.semaphore_signal(
            barrier_sem, inc=1, device_id=nbr,
            device_id_type=pl.DeviceIdType.MESH,
        )

        acc_val = None
        for i in range(N_CHUNKS):
            chunk_copy(x_hbm, buf, copy_sems, i).wait()
            chunk_max = jnp.max(buf[i], axis=0, keepdims=True)
            acc_val = chunk_max if acc_val is None else jnp.maximum(acc_val, chunk_max)
        acc[:, :] = acc_val

        pl.semaphore_wait(barrier_sem, 1)

        rdma = pltpu.make_async_remote_copy(
            src_ref=acc,
            dst_ref=recv_buf,
            send_sem=send_sem,
            recv_sem=recv_sem,
            device_id=nbr,
            device_id_type=pl.DeviceIdType.MESH,
        )
        rdma.start()
        rdma.wait_recv()

        out_vmem[:, :] = jnp.maximum(acc[:, :], recv_buf[:, :])
        out_cp = pltpu.make_async_copy(out_vmem, out_hbm, out_copy_sem)
        out_cp.start()
        rdma.wait_send()
        out_cp.wait()

    return pl.pallas_call(
        body,
        out_shape=jax.ShapeDtypeStruct((1, n), x.dtype),
        in_specs=[pl.BlockSpec(memory_space=pltpu.MemorySpace.HBM)],
        out_specs=pl.BlockSpec(memory_space=pltpu.MemorySpace.HBM),
        scratch_shapes=[
            pltpu.VMEM((N_CHUNKS, m // N_CHUNKS, n), x.dtype),
            pltpu.VMEM((1, n), x.dtype),
            pltpu.VMEM((1, n), x.dtype),
            pltpu.VMEM((1, n), x.dtype),
            pltpu.SemaphoreType.DMA((N_CHUNKS,)),
            pltpu.SemaphoreType.DMA,
            pltpu.SemaphoreType.DMA,
            pltpu.SemaphoreType.DMA,
        ],
        compiler_params=pltpu.CompilerParams(collective_id=0),
    )(pltpu.with_memory_space_constraint(x, pltpu.MemorySpace.HBM))
